# baseline (speedup 1.0000x reference)
import functools

import jax
import jax.numpy as jnp
from jax import lax
from jax.experimental import pallas as pl
from jax.experimental.pallas import tpu as pltpu
from jax.experimental.pallas import tpu_sc as plsc

VOCAB = 1_000_000
EMBED_DIM = 32
BATCH = 16384
HIST = 50

NUM_WORKERS = 32
TOTAL = BATCH * HIST
PER_WORKER = TOTAL // NUM_WORKERS
GATHER = 128
NUM_GATHERS = PER_WORKER // GATHER
CHUNKS_PER_BLK = 10
BLK_ROWS = CHUNKS_PER_BLK * GATHER
NUM_BLOCKS = NUM_GATHERS // CHUNKS_PER_BLK

_mesh = plsc.VectorSubcoreMesh(core_axis_name="c", subcore_axis_name="s")


@functools.partial(
    pl.kernel,
    mesh=_mesh,
    out_type=jax.ShapeDtypeStruct((TOTAL, EMBED_DIM), jnp.float32),
    compiler_params=pltpu.CompilerParams(use_tc_tiling_on_sc=False),
    scratch_types=[
        pltpu.VMEM((NUM_GATHERS, GATHER), jnp.int32),
        pltpu.VMEM((BLK_ROWS, EMBED_DIM), jnp.float32),
        pltpu.VMEM((BLK_ROWS, EMBED_DIM), jnp.float32),
        [pltpu.SemaphoreType.DMA] * 2,
        [pltpu.SemaphoreType.DMA] * 2,
    ],
)
def _embed_gather(x_hbm, table_hbm, out_hbm, idx_v, buf0, buf1, semg, semw):
    wid = lax.axis_index("s") * 2 + lax.axis_index("c")
    base = wid * PER_WORKER
    bufs = (buf0, buf1)
    pltpu.sync_copy(x_hbm.at[wid], idx_v)

    def gathers(blk, p, start=True):
        mk = pltpu.async_copy if start else pltpu.make_async_copy
        return [
            mk(table_hbm.at[idx_v.at[blk * CHUNKS_PER_BLK + c]],
               bufs[p].at[pl.ds(c * GATHER, GATHER)],
               semg[p])
            for c in range(CHUNKS_PER_BLK)
        ]

    def writeback(blk, p, start=True):
        mk = pltpu.async_copy if start else pltpu.make_async_copy
        return mk(bufs[p], out_hbm.at[pl.ds(base + blk * BLK_ROWS, BLK_ROWS)],
                  semw[p])

    gathers(0, 0)
    gathers(1, 1)

    def body(pi, carry):
        b0 = 2 * pi
        for d in gathers(b0, 0, start=False):
            d.wait()
        writeback(b0, 0)

        for d in gathers(b0 + 1, 1, start=False):
            d.wait()

        writeback(b0, 0, start=False).wait()
        gathers(b0 + 2, 0)

        writeback(b0 + 1, 1)
        writeback(b0 + 1, 1, start=False).wait()
        gathers(b0 + 3, 1)
        return carry

    lax.fori_loop(0, NUM_BLOCKS // 2 - 1, body, 0)

    for d in gathers(NUM_BLOCKS - 2, 0, start=False):
        d.wait()
    writeback(NUM_BLOCKS - 2, 0)
    for d in gathers(NUM_BLOCKS - 1, 1, start=False):
        d.wait()
    writeback(NUM_BLOCKS - 1, 1)
    writeback(NUM_BLOCKS - 2, 0, start=False).wait()
    writeback(NUM_BLOCKS - 1, 1, start=False).wait()


def kernel(x, table):
    xr = x.reshape(NUM_WORKERS, NUM_GATHERS, GATHER)
    out = _embed_gather(xr, table)
    return out.reshape(BATCH, HIST, EMBED_DIM)


# speedup vs baseline: 1.7420x; 1.7420x over previous
import functools

import jax
import jax.numpy as jnp
from jax import lax
from jax.experimental import pallas as pl
from jax.experimental.pallas import tpu as pltpu
from jax.experimental.pallas import tpu_sc as plsc

VOCAB = 1_000_000
EMBED_DIM = 32
BATCH = 16384
HIST = 50

NUM_WORKERS = 32
TOTAL = BATCH * HIST
PER_WORKER = TOTAL // NUM_WORKERS
GATHER = 128
NUM_GATHERS = PER_WORKER // GATHER
CHUNKS_PER_BLK = 10
BLK_ROWS = CHUNKS_PER_BLK * GATHER
NUM_BLOCKS = NUM_GATHERS // CHUNKS_PER_BLK

_mesh = plsc.VectorSubcoreMesh(core_axis_name="c", subcore_axis_name="s")


@functools.partial(
    pl.kernel,
    mesh=_mesh,
    out_type=jax.ShapeDtypeStruct((TOTAL, EMBED_DIM), jnp.float32),
    compiler_params=pltpu.CompilerParams(use_tc_tiling_on_sc=False),
    scratch_types=[
        pltpu.VMEM((PER_WORKER,), jnp.int32),
        pltpu.VMEM((BLK_ROWS, EMBED_DIM), jnp.float32),
        pltpu.VMEM((BLK_ROWS, EMBED_DIM), jnp.float32),
        [pltpu.SemaphoreType.DMA] * 2,
        [pltpu.SemaphoreType.DMA] * 2,
    ],
)
def _embed_gather(x_hbm, table_hbm, out_hbm, idx_v, buf0, buf1, semg, semw):
    wid = lax.axis_index("s") * 2 + lax.axis_index("c")
    base = wid * PER_WORKER
    bufs = (buf0, buf1)
    pltpu.sync_copy(x_hbm.at[pl.ds(base, PER_WORKER)], idx_v)

    def gathers(blk, p, start=True):
        mk = pltpu.async_copy if start else pltpu.make_async_copy
        return [
            mk(table_hbm.at[idx_v.at[pl.ds((blk * CHUNKS_PER_BLK + c) * GATHER,
                                           GATHER)]],
               bufs[p].at[pl.ds(c * GATHER, GATHER)],
               semg[p])
            for c in range(CHUNKS_PER_BLK)
        ]

    def writeback(blk, p, start=True):
        mk = pltpu.async_copy if start else pltpu.make_async_copy
        return mk(bufs[p], out_hbm.at[pl.ds(base + blk * BLK_ROWS, BLK_ROWS)],
                  semw[p])

    gathers(0, 0)
    gathers(1, 1)

    def body(pi, carry):
        b0 = 2 * pi
        for d in gathers(b0, 0, start=False):
            d.wait()
        writeback(b0, 0)

        for d in gathers(b0 + 1, 1, start=False):
            d.wait()

        writeback(b0, 0, start=False).wait()
        gathers(b0 + 2, 0)

        writeback(b0 + 1, 1)
        writeback(b0 + 1, 1, start=False).wait()
        gathers(b0 + 3, 1)
        return carry

    lax.fori_loop(0, NUM_BLOCKS // 2 - 1, body, 0)

    for d in gathers(NUM_BLOCKS - 2, 0, start=False):
        d.wait()
    writeback(NUM_BLOCKS - 2, 0)
    for d in gathers(NUM_BLOCKS - 1, 1, start=False):
        d.wait()
    writeback(NUM_BLOCKS - 1, 1)
    writeback(NUM_BLOCKS - 2, 0, start=False).wait()
    writeback(NUM_BLOCKS - 1, 1, start=False).wait()


def kernel(x, table):
    idx = x.T.reshape(TOTAL)
    out = _embed_gather(idx, table)
    return out.reshape(HIST, BATCH, EMBED_DIM).transpose(1, 0, 2)
